# baseline (speedup 1.0000x reference)
import dataclasses
import functools

import jax
import jax.numpy as jnp
from jax import lax
from jax.experimental import pallas as pl
from jax.experimental.pallas import tpu as pltpu
from jax.experimental.pallas import tpu_sc as plsc

B = 1024
H = 200
V = 1000
NC = 2
NS = 16
L = 16
NW = NC * NS
RPW = B // NW
PAIRS = RPW // 2
PVEC = 2 * H // L
INV_H = 1.0 / H

_cp = pltpu.CompilerParams(has_side_effects=True)
if "needs_layout_passes" in pltpu.CompilerParams.__dataclass_fields__:
    _cp = dataclasses.replace(_cp, needs_layout_passes=False)


def _body(a_hbm, out_hbm, a_v, hist_v, sem):
    wid = lax.axis_index("s") * NC + lax.axis_index("c")

    in_cp = pltpu.async_copy(a_hbm.at[pl.ds(wid * RPW * H, RPW * H)], a_v, sem)

    zeros = jnp.zeros((L,), jnp.float32)

    @plsc.parallel_loop(0, RPW * V // L, unroll=10)
    def _zero(j):
        hist_v[pl.ds(j * L, L)] = zeros

    in_cp.wait()

    iota = lax.iota(jnp.int32, L)
    straddle = jnp.where(iota >= 8, V, 0).astype(jnp.int32)
    second = jnp.full((L,), V, jnp.int32)
    val = jnp.full((L,), INV_H, jnp.float32)

    @plsc.parallel_loop(0, PAIRS)
    def _pair(p):
        base = jnp.broadcast_to(p * (2 * V), (L,)).astype(jnp.int32)
        for j in range(PVEC):
            idx = a_v[pl.ds(p * (2 * H) + j * L, L)]
            if j < H // L:
                off = base
            elif j == H // L:
                off = base + straddle
            else:
                off = base + second
            plsc.addupdate_scatter(hist_v, [idx + off], val)

    pltpu.sync_copy(hist_v, out_hbm.at[pl.ds(wid * RPW * V, RPW * V)])


@jax.jit
def kernel(assignments):
    mesh = plsc.VectorSubcoreMesh(
        core_axis_name="c", subcore_axis_name="s", num_cores=NC, num_subcores=NS
    )
    run = pl.kernel(
        _body,
        out_type=jax.ShapeDtypeStruct((B * V,), jnp.float32),
        mesh=mesh,
        scratch_types=[
            pltpu.VMEM((RPW * H,), jnp.int32),
            pltpu.VMEM((RPW * V,), jnp.float32),
            pltpu.SemaphoreType.DMA,
        ],
        compiler_params=_cp,
    )
    return run(assignments.reshape(B * H)).reshape(B, V)


# speedup vs baseline: 1.0998x; 1.0998x over previous
import dataclasses
import functools

import jax
import jax.numpy as jnp
from jax import lax
from jax.experimental import pallas as pl
from jax.experimental.pallas import tpu as pltpu
from jax.experimental.pallas import tpu_sc as plsc

B = 1024
H = 200
V = 1000
NC = 2
NS = 16
L = 16
NW = NC * NS
RPW = B // NW
FULL = H // L
INV_H = 1.0 / H

_cp = pltpu.CompilerParams(has_side_effects=True)
if "needs_layout_passes" in pltpu.CompilerParams.__dataclass_fields__:
    _cp = dataclasses.replace(_cp, needs_layout_passes=False)


def _body(a_hbm, out_hbm, a_v, hist_v, sem):
    wid = lax.axis_index("s") * NC + lax.axis_index("c")
    row0 = wid * RPW

    in_cp = pltpu.async_copy(a_hbm.at[pl.ds(row0, RPW)], a_v, sem)

    zeros = jnp.zeros((L,), jnp.float32)

    @plsc.parallel_loop(0, RPW)
    def _zero(r):
        for j in range(V // L):
            hist_v[r, pl.ds(j * L, L)] = zeros
        hist_v[r, pl.ds(V - L, L)] = zeros

    in_cp.wait()

    iota = lax.iota(jnp.int32, L)
    tail_mask = iota >= 8
    val = jnp.full((L,), INV_H, jnp.float32)

    @plsc.parallel_loop(0, RPW)
    def _row(r):
        row = jnp.broadcast_to(r, (L,)).astype(jnp.int32)
        for j in range(FULL):
            idx = a_v[r, pl.ds(j * L, L)]
            plsc.addupdate_scatter(hist_v, [row, idx], val)
        idx = a_v[r, pl.ds(H - L, L)]
        plsc.addupdate_scatter(hist_v, [row, idx], val, mask=tail_mask)

    pltpu.sync_copy(hist_v, out_hbm.at[pl.ds(row0, RPW)])


@jax.jit
def kernel(assignments):
    mesh = plsc.VectorSubcoreMesh(
        core_axis_name="c", subcore_axis_name="s", num_cores=NC, num_subcores=NS
    )
    run = pl.kernel(
        _body,
        out_type=jax.ShapeDtypeStruct((B, V), jnp.float32),
        mesh=mesh,
        scratch_types=[
            pltpu.VMEM((RPW, H), jnp.int32),
            pltpu.VMEM((RPW, V), jnp.float32),
            pltpu.SemaphoreType.DMA,
        ],
        compiler_params=_cp,
    )
    return run(assignments)
